# Initial kernel scaffold; baseline (speedup 1.0000x reference)
import functools

import jax
import jax.numpy as jnp
from jax import lax
from jax.experimental import pallas as pl
from jax.experimental.pallas import tpu as pltpu
from jax.experimental.pallas import tpu_sc as plsc

N_NODES = 10000
D = 128
N_EDGES = 320000

NC = 2
NS = 16
NW = NC * NS
LANES = 16

NP = 10240
CH = 128
K = -(-N_EDGES // (NW * CH))
EP = NW * K * CH
ROWS_PER_TILE = NP // NS
ZR = 64


def _sc_body(x_hbm, src_hbm, dst_hbm, acc_out, deg_out,
             src_v, dst_v, rows_v, zbuf_v, hist_v, acc_sh, sem0, sem1,
             *, with_deg):
    c = lax.axis_index("c")
    s = lax.axis_index("s")
    wid = c * NS + s
    sems = (sem0, sem1)

    zeros16 = jnp.zeros((LANES,), jnp.float32)

    @pl.loop(0, ZR)
    def _(r):
        @pl.loop(0, D // LANES)
        def _(g):
            zbuf_v[r, pl.ds(g * LANES, LANES)] = zeros16

    @pl.loop(0, ROWS_PER_TILE // ZR)
    def _(k):
        pltpu.sync_copy(zbuf_v, acc_sh.at[pl.ds(s * ROWS_PER_TILE + k * ZR, ZR)])

    if with_deg:
        @pl.loop(0, NP // LANES)
        def _(g):
            hist_v[pl.ds(g * LANES, LANES)] = zeros16

    pltpu.sync_copy(src_hbm.at[wid], src_v)
    pltpu.sync_copy(dst_hbm.at[wid], dst_v)

    plsc.subcore_barrier()

    ones16 = jnp.full((LANES,), 1.0, jnp.float32)

    pltpu.async_copy(x_hbm.at[src_v.at[0]], rows_v.at[0], sems[0])

    @pl.loop(0, K + (K % 2), step=2)
    def _(j0):
        for b in range(2):
            j = j0 + b

            @pl.when(j + 1 < K)
            def _():
                pltpu.async_copy(x_hbm.at[src_v.at[j + 1]],
                                 rows_v.at[1 - b], sems[1 - b])

            @pl.when(j < K)
            def _():
                pltpu.make_async_copy(x_hbm.at[src_v.at[j]],
                                      rows_v.at[b], sems[b]).wait()
                pltpu.sync_copy(rows_v.at[b], acc_sh.at[dst_v.at[j]], add=True)
                if with_deg:
                    @pl.loop(0, CH // LANES)
                    def _(g):
                        idx16 = dst_v[j, pl.ds(g * LANES, LANES)]
                        plsc.addupdate_scatter(hist_v, [idx16], ones16)

    plsc.subcore_barrier()

    pltpu.sync_copy(acc_sh.at[pl.ds(s * ROWS_PER_TILE, ROWS_PER_TILE)],
                    acc_out.at[c, pl.ds(s * ROWS_PER_TILE, ROWS_PER_TILE)])
    if with_deg:
        pltpu.sync_copy(hist_v, deg_out.at[wid])


def _make_sc_kernel(with_deg):
    mesh = plsc.VectorSubcoreMesh(core_axis_name="c", subcore_axis_name="s")
    out_type = [jax.ShapeDtypeStruct((NC, NP, D), jnp.float32)]
    if with_deg:
        out_type.append(jax.ShapeDtypeStruct((NW, NP), jnp.float32))
    scratch = [
        pltpu.VMEM((K, CH), jnp.int32),
        pltpu.VMEM((K, CH), jnp.int32),
        pltpu.VMEM((2, CH, D), jnp.float32),
        pltpu.VMEM((ZR, D), jnp.float32),
        pltpu.VMEM((NP,), jnp.float32),
        pltpu.VMEM_SHARED((NP, D), jnp.float32),
        pltpu.SemaphoreType.DMA,
        pltpu.SemaphoreType.DMA,
    ]

    def body(x_hbm, src_hbm, dst_hbm, *rest):
        if with_deg:
            acc_out, deg_out = rest[0], rest[1]
            scr = rest[2:]
        else:
            acc_out, deg_out = rest[0], None
            scr = rest[1:]
        _sc_body(x_hbm, src_hbm, dst_hbm, acc_out, deg_out, *scr,
                 with_deg=with_deg)

    return pl.kernel(body, out_type=out_type, mesh=mesh, scratch_types=scratch,
                     name="sage_sc_deg" if with_deg else "sage_sc")


_sc_layer1 = _make_sc_kernel(True)
_sc_layer2 = _make_sc_kernel(False)

BR = 512


def _tc_body(acc_ref, degp_ref, x_ref, wl_ref, bl_ref, wr_ref, out_ref, *, elu):
    deg = jnp.sum(degp_ref[...], axis=0)
    deginv = 1.0 / jnp.maximum(deg, 1.0)
    agg = (acc_ref[0] + acc_ref[1]) * deginv[:, None]
    h = (jnp.dot(agg, wl_ref[...], preferred_element_type=jnp.float32)
         + bl_ref[...]
         + jnp.dot(x_ref[...], wr_ref[...], preferred_element_type=jnp.float32))
    if elu:
        h = jnp.where(h > 0, h, jnp.expm1(h))
    out_ref[...] = h


def _tc_layer(acc, degp, x, wlT, bl, wrT, elu):
    grid = (NP // BR,)
    return pl.pallas_call(
        functools.partial(_tc_body, elu=elu),
        grid=grid,
        in_specs=[
            pl.BlockSpec((NC, BR, D), lambda i: (0, i, 0)),
            pl.BlockSpec((NW, BR), lambda i: (0, i)),
            pl.BlockSpec((BR, D), lambda i: (i, 0)),
            pl.BlockSpec((D, D), lambda i: (0, 0)),
            pl.BlockSpec((1, D), lambda i: (0, 0)),
            pl.BlockSpec((D, D), lambda i: (0, 0)),
        ],
        out_specs=pl.BlockSpec((BR, D), lambda i: (i, 0)),
        out_shape=jax.ShapeDtypeStruct((NP, D), jnp.float32),
    )(acc, degp, x, wlT, bl, wrT)


def kernel(x, edge_index, W1l, b1l, W1r, W2l, b2l, W2r):
    src = edge_index[0].astype(jnp.int32)
    dst = edge_index[1].astype(jnp.int32)
    pad = EP - N_EDGES
    src_r = jnp.concatenate(
        [src, jnp.zeros((pad,), jnp.int32)]).reshape(NW, K, CH)
    dst_r = jnp.concatenate(
        [dst, jnp.full((pad,), N_NODES, jnp.int32)]).reshape(NW, K, CH)
    x_p = jnp.pad(x, ((0, NP - N_NODES), (0, 0)))

    acc1, degp = _sc_layer1(x_p, src_r, dst_r)
    h = _tc_layer(acc1, degp, x_p, W1l.T, b1l[None, :], W1r.T, elu=True)
    acc2, = _sc_layer2(h, src_r, dst_r)
    out = _tc_layer(acc2, degp, h, W2l.T, b2l[None, :], W2r.T, elu=False)
    return out[:N_NODES]


# speedup vs baseline: 8.6157x; 8.6157x over previous
import functools

import jax
import jax.numpy as jnp
from jax import lax
from jax.experimental import pallas as pl
from jax.experimental.pallas import tpu as pltpu
from jax.experimental.pallas import tpu_sc as plsc

N_NODES = 10000
D = 128
N_EDGES = 320000

NC = 2
NS = 16
NW = NC * NS
LANES = 16

NP = 10240
CH = 64
K = -(-N_EDGES // (NW * CH))
EP = NW * K * CH
ROWS_PER_TILE = NP // NS


def _sc_body(x_hbm, src_hbm, dst_hbm, acc_out, deg_out,
             src_v, dst_v, rows_v, hist_v, acc_sh, sem0, sem1,
             *, with_deg):
    c = lax.axis_index("c")
    s = lax.axis_index("s")
    wid = c * NS + s
    sems = (sem0, sem1)

    zeros16 = jnp.zeros((LANES,), jnp.float32)

    @pl.loop(0, 2 * CH)
    def _(r):
        @pl.loop(0, D // LANES)
        def _(g):
            rows_v[r // CH, r % CH, pl.ds(g * LANES, LANES)] = zeros16

    @pl.loop(0, ROWS_PER_TILE // CH)
    def _(k):
        pltpu.sync_copy(rows_v.at[0],
                        acc_sh.at[pl.ds(s * ROWS_PER_TILE + k * CH, CH)])

    if with_deg:
        @pl.loop(0, NP // LANES)
        def _(g):
            hist_v[pl.ds(g * LANES, LANES)] = zeros16

    pltpu.sync_copy(src_hbm.at[wid], src_v)
    pltpu.sync_copy(dst_hbm.at[wid], dst_v)

    plsc.subcore_barrier()

    ones16 = jnp.full((LANES,), 1.0, jnp.float32)

    pltpu.async_copy(x_hbm.at[src_v.at[0]], rows_v.at[0], sems[0])

    @pl.loop(0, K + (K % 2), step=2)
    def _(j0):
        for b in range(2):
            j = j0 + b

            @pl.when(j + 1 < K)
            def _():
                pltpu.async_copy(x_hbm.at[src_v.at[j + 1]],
                                 rows_v.at[1 - b], sems[1 - b])

            @pl.when(j < K)
            def _():
                pltpu.make_async_copy(x_hbm.at[src_v.at[j]],
                                      rows_v.at[b], sems[b]).wait()
                pltpu.sync_copy(rows_v.at[b], acc_sh.at[dst_v.at[j]], add=True)
                if with_deg:
                    @pl.loop(0, CH // LANES)
                    def _(g):
                        idx16 = dst_v[j, pl.ds(g * LANES, LANES)]
                        plsc.addupdate_scatter(hist_v, [idx16], ones16)

    plsc.subcore_barrier()

    pltpu.sync_copy(acc_sh.at[pl.ds(s * ROWS_PER_TILE, ROWS_PER_TILE)],
                    acc_out.at[c, pl.ds(s * ROWS_PER_TILE, ROWS_PER_TILE)])
    if with_deg:
        pltpu.sync_copy(hist_v, deg_out.at[wid])


def _make_sc_kernel(with_deg):
    mesh = plsc.VectorSubcoreMesh(core_axis_name="c", subcore_axis_name="s")
    out_type = [pltpu.HBM((NC, NP, D), jnp.float32)]
    if with_deg:
        out_type.append(pltpu.HBM((NW, NP), jnp.float32))
    scratch = [
        pltpu.VMEM((K, CH), jnp.int32),
        pltpu.VMEM((K, CH), jnp.int32),
        pltpu.VMEM((2, CH, D), jnp.float32),
    ]
    if with_deg:
        scratch.append(pltpu.VMEM((NP,), jnp.float32))
    scratch += [
        pltpu.VMEM_SHARED((NP, D), jnp.float32),
        pltpu.SemaphoreType.DMA,
        pltpu.SemaphoreType.DMA,
    ]

    def body(x_hbm, src_hbm, dst_hbm, *rest):
        if with_deg:
            acc_out, deg_out = rest[0], rest[1]
            src_v, dst_v, rows_v, hist_v, acc_sh, sem0, sem1 = rest[2:]
        else:
            acc_out, deg_out = rest[0], None
            src_v, dst_v, rows_v, acc_sh, sem0, sem1 = rest[1:]
            hist_v = None
        _sc_body(x_hbm, src_hbm, dst_hbm, acc_out, deg_out,
                 src_v, dst_v, rows_v, hist_v, acc_sh, sem0, sem1,
                 with_deg=with_deg)

    return pl.kernel(body, out_type=out_type, mesh=mesh, scratch_types=scratch,
                     compiler_params=pltpu.CompilerParams(
                         needs_layout_passes=False,
                         use_tc_tiling_on_sc=False),
                     name="sage_sc_deg" if with_deg else "sage_sc")


_sc_layer1 = _make_sc_kernel(True)
_sc_layer2 = _make_sc_kernel(False)

BR = 512


def _tc_body(acc_ref, degp_ref, x_ref, wl_ref, bl_ref, wr_ref, out_ref, *, elu):
    deg = jnp.sum(degp_ref[...], axis=0)
    deginv = 1.0 / jnp.maximum(deg, 1.0)
    agg = (acc_ref[0] + acc_ref[1]) * deginv[:, None]
    h = (jnp.dot(agg, wl_ref[...], preferred_element_type=jnp.float32)
         + bl_ref[...]
         + jnp.dot(x_ref[...], wr_ref[...], preferred_element_type=jnp.float32))
    if elu:
        h = jnp.where(h > 0, h, jnp.exp(jnp.minimum(h, 0.0)) - 1.0)
    out_ref[...] = h


def _tc_layer(acc, degp, x, wlT, bl, wrT, elu):
    grid = (NP // BR,)
    return pl.pallas_call(
        functools.partial(_tc_body, elu=elu),
        grid=grid,
        in_specs=[
            pl.BlockSpec((NC, BR, D), lambda i: (0, i, 0)),
            pl.BlockSpec((NW, BR), lambda i: (0, i)),
            pl.BlockSpec((BR, D), lambda i: (i, 0)),
            pl.BlockSpec((D, D), lambda i: (0, 0)),
            pl.BlockSpec((1, D), lambda i: (0, 0)),
            pl.BlockSpec((D, D), lambda i: (0, 0)),
        ],
        out_specs=pl.BlockSpec((BR, D), lambda i: (i, 0)),
        out_shape=jax.ShapeDtypeStruct((NP, D), jnp.float32),
    )(acc, degp, x, wlT, bl, wrT)


def kernel(x, edge_index, W1l, b1l, W1r, W2l, b2l, W2r):
    src = edge_index[0].astype(jnp.int32)
    dst = edge_index[1].astype(jnp.int32)
    pad = EP - N_EDGES
    src_r = jnp.concatenate(
        [src, jnp.zeros((pad,), jnp.int32)]).reshape(NW, K, CH)
    dst_r = jnp.concatenate(
        [dst, jnp.full((pad,), N_NODES, jnp.int32)]).reshape(NW, K, CH)
    x_p = jnp.pad(x, ((0, NP - N_NODES), (0, 0)))

    acc1, degp = _sc_layer1(x_p, src_r, dst_r)
    h = _tc_layer(acc1, degp, x_p, W1l.T, b1l[None, :], W1r.T, elu=True)
    acc2, = _sc_layer2(h, src_r, dst_r)
    out = _tc_layer(acc2, degp, h, W2l.T, b2l[None, :], W2r.T, elu=False)
    return out[:N_NODES]


# speedup vs baseline: 9.6285x; 1.1176x over previous
import functools

import jax
import jax.numpy as jnp
from jax import lax
from jax.experimental import pallas as pl
from jax.experimental.pallas import tpu as pltpu
from jax.experimental.pallas import tpu_sc as plsc

N_NODES = 10000
D = 128
N_EDGES = 320000

NC = 2
NS = 16
NW = NC * NS
LANES = 16

NP = 10240
CH = 40
K = N_EDGES // (NW * CH)
ROWS_PER_TILE = NP // NS
WB_FULL = N_NODES // NS


def _sc_body(x_hbm, src_hbm, dst_hbm, acc_out, deg_out,
             src_v, dst_v, rows_v, hist_v, acc_sh, sem0, sem1,
             *, with_deg):
    c = lax.axis_index("c")
    s = lax.axis_index("s")
    wid = c * NS + s
    sems = (sem0, sem1)

    zeros16 = jnp.zeros((LANES,), jnp.float32)

    @pl.loop(0, 2 * CH)
    def _(r):
        @pl.loop(0, D // LANES)
        def _(g):
            rows_v[r // CH, r % CH, pl.ds(g * LANES, LANES)] = zeros16

    @pl.loop(0, ROWS_PER_TILE // CH)
    def _(k):
        pltpu.sync_copy(rows_v.at[0],
                        acc_sh.at[pl.ds(s * ROWS_PER_TILE + k * CH, CH)])

    if with_deg:
        @pl.loop(0, N_NODES // LANES)
        def _(g):
            hist_v[pl.ds(g * LANES, LANES)] = zeros16

    pltpu.sync_copy(src_hbm.at[wid], src_v)
    pltpu.sync_copy(dst_hbm.at[wid], dst_v)

    plsc.subcore_barrier()

    ones16 = jnp.full((LANES,), 1.0, jnp.float32)

    pltpu.async_copy(x_hbm.at[src_v.at[0]], rows_v.at[0], sems[0])

    @pl.loop(0, K, step=2)
    def _(j0):
        for b in range(2):
            j = j0 + b

            @pl.when(j + 1 < K)
            def _():
                pltpu.async_copy(x_hbm.at[src_v.at[j + 1]],
                                 rows_v.at[1 - b], sems[1 - b])

            pltpu.make_async_copy(x_hbm.at[src_v.at[j]],
                                  rows_v.at[b], sems[b]).wait()
            pltpu.sync_copy(rows_v.at[b], acc_sh.at[dst_v.at[j]], add=True)
            if with_deg:
                for g in range(CH // LANES):
                    idx16 = dst_v[j, pl.ds(g * LANES, LANES)]
                    plsc.addupdate_scatter(hist_v, [idx16], ones16)
                tail = CH - (CH // LANES) * LANES
                if tail:
                    idx16 = dst_v[j, pl.ds(CH - LANES, LANES)]
                    mask = lax.iota(jnp.int32, LANES) >= (LANES - tail)
                    plsc.addupdate_scatter(hist_v, [idx16], ones16, mask=mask)

    plsc.subcore_barrier()

    pltpu.sync_copy(acc_sh.at[pl.ds(s * WB_FULL, WB_FULL)],
                    acc_out.at[c, pl.ds(s * WB_FULL, WB_FULL)])

    if with_deg:
        pltpu.sync_copy(hist_v, deg_out.at[wid])


def _make_sc_kernel(with_deg):
    mesh = plsc.VectorSubcoreMesh(core_axis_name="c", subcore_axis_name="s")
    out_type = [pltpu.HBM((NC, N_NODES, D), jnp.float32)]
    if with_deg:
        out_type.append(pltpu.HBM((NW, N_NODES), jnp.float32))
    scratch = [
        pltpu.VMEM((K, CH), jnp.int32),
        pltpu.VMEM((K, CH), jnp.int32),
        pltpu.VMEM((2, CH, D), jnp.float32),
    ]
    if with_deg:
        scratch.append(pltpu.VMEM((N_NODES,), jnp.float32))
    scratch += [
        pltpu.VMEM_SHARED((NP, D), jnp.float32),
        pltpu.SemaphoreType.DMA,
        pltpu.SemaphoreType.DMA,
    ]

    def body(x_hbm, src_hbm, dst_hbm, *rest):
        if with_deg:
            acc_out, deg_out = rest[0], rest[1]
            src_v, dst_v, rows_v, hist_v, acc_sh, sem0, sem1 = rest[2:]
        else:
            acc_out, deg_out = rest[0], None
            src_v, dst_v, rows_v, acc_sh, sem0, sem1 = rest[1:]
            hist_v = None
        _sc_body(x_hbm, src_hbm, dst_hbm, acc_out, deg_out,
                 src_v, dst_v, rows_v, hist_v, acc_sh, sem0, sem1,
                 with_deg=with_deg)

    return pl.kernel(body, out_type=out_type, mesh=mesh, scratch_types=scratch,
                     compiler_params=pltpu.CompilerParams(
                         needs_layout_passes=False,
                         use_tc_tiling_on_sc=False),
                     name="sage_sc_deg" if with_deg else "sage_sc")


_sc_layer1 = _make_sc_kernel(True)
_sc_layer2 = _make_sc_kernel(False)

BR = 400


def _deg_body(degp_ref, out_ref):
    deg = jnp.sum(degp_ref[...], axis=0)
    out_ref[...] = (1.0 / jnp.maximum(deg, 1.0))[:, None]


_deg_reduce = pl.pallas_call(
    _deg_body,
    out_shape=jax.ShapeDtypeStruct((N_NODES, 1), jnp.float32),
)


def _tc_body(acc_ref, dinv_ref, x_ref, wl_ref, bl_ref, wr_ref, out_ref, *, elu):
    agg = (acc_ref[0] + acc_ref[1]) * dinv_ref[...]
    h = (jnp.dot(agg, wl_ref[...], preferred_element_type=jnp.float32)
         + bl_ref[...]
         + jnp.dot(x_ref[...], wr_ref[...], preferred_element_type=jnp.float32))
    if elu:
        h = jnp.where(h > 0, h, jnp.exp(jnp.minimum(h, 0.0)) - 1.0)
    out_ref[...] = h


def _tc_layer(acc, dinv, x, wlT, bl, wrT, elu):
    grid = (N_NODES // BR,)
    return pl.pallas_call(
        functools.partial(_tc_body, elu=elu),
        grid=grid,
        in_specs=[
            pl.BlockSpec((NC, BR, D), lambda i: (0, i, 0)),
            pl.BlockSpec((BR, 1), lambda i: (i, 0)),
            pl.BlockSpec((BR, D), lambda i: (i, 0)),
            pl.BlockSpec((D, D), lambda i: (0, 0)),
            pl.BlockSpec((1, D), lambda i: (0, 0)),
            pl.BlockSpec((D, D), lambda i: (0, 0)),
        ],
        out_specs=pl.BlockSpec((BR, D), lambda i: (i, 0)),
        out_shape=jax.ShapeDtypeStruct((N_NODES, D), jnp.float32),
    )(acc, dinv, x, wlT, bl, wrT)


def kernel(x, edge_index, W1l, b1l, W1r, W2l, b2l, W2r):
    src_r = edge_index[0].astype(jnp.int32).reshape(NW, K, CH)
    dst_r = edge_index[1].astype(jnp.int32).reshape(NW, K, CH)

    acc1, degp = _sc_layer1(x, src_r, dst_r)
    dinv = _deg_reduce(degp)
    h = _tc_layer(acc1, dinv, x, W1l.T, b1l[None, :], W1r.T, elu=True)
    acc2, = _sc_layer2(h, src_r, dst_r)
    out = _tc_layer(acc2, dinv, h, W2l.T, b2l[None, :], W2r.T, elu=False)
    return out


# speedup vs baseline: 12.0480x; 1.2513x over previous
import functools

import jax
import jax.numpy as jnp
from jax import lax
from jax.experimental import pallas as pl
from jax.experimental.pallas import tpu as pltpu
from jax.experimental.pallas import tpu_sc as plsc

N_NODES = 10000
D = 128
N_EDGES = 320000

NC = 2
NS = 16
NW = NC * NS
LANES = 16

CH = 80
K = N_EDGES // (NW * CH)
WB_FULL = N_NODES // NS
ZC = 25


def _sc_body(x_hbm, src_hbm, dst_hbm, acc_out, deg_out,
             src_v, dst_v, rows_v, hist_v, acc_sh, sem0, sem1,
             *, with_deg):
    c = lax.axis_index("c")
    s = lax.axis_index("s")
    wid = c * NS + s
    sems = (sem0, sem1)

    zeros16 = jnp.zeros((LANES,), jnp.float32)

    @pl.loop(0, 2 * CH)
    def _(r):
        @pl.loop(0, D // LANES)
        def _(g):
            rows_v[r // CH, r % CH, pl.ds(g * LANES, LANES)] = zeros16

    @pl.loop(0, WB_FULL // ZC)
    def _(k):
        pltpu.sync_copy(rows_v.at[0, pl.ds(0, ZC)],
                        acc_sh.at[pl.ds(s * WB_FULL + k * ZC, ZC)])

    if with_deg:
        @pl.loop(0, N_NODES // LANES)
        def _(g):
            hist_v[pl.ds(g * LANES, LANES)] = zeros16

    pltpu.sync_copy(src_hbm.at[wid], src_v)
    pltpu.sync_copy(dst_hbm.at[wid], dst_v)

    plsc.subcore_barrier()

    ones16 = jnp.full((LANES,), 1.0, jnp.float32)

    pltpu.async_copy(x_hbm.at[src_v.at[0]], rows_v.at[0], sems[0])

    @pl.loop(0, K + (K % 2), step=2)
    def _(j0):
        for b in range(2):
            j = j0 + b

            @pl.when(j + 1 < K)
            def _():
                pltpu.async_copy(x_hbm.at[src_v.at[j + 1]],
                                 rows_v.at[1 - b], sems[1 - b])

            @pl.when(j < K)
            def _():
                pltpu.make_async_copy(x_hbm.at[src_v.at[j]],
                                      rows_v.at[b], sems[b]).wait()
                pltpu.sync_copy(rows_v.at[b], acc_sh.at[dst_v.at[j]], add=True)
                if with_deg:
                    for g in range(CH // LANES):
                        idx16 = dst_v[j, pl.ds(g * LANES, LANES)]
                        plsc.addupdate_scatter(hist_v, [idx16], ones16)

    plsc.subcore_barrier()

    pltpu.sync_copy(acc_sh.at[pl.ds(s * WB_FULL, WB_FULL)],
                    acc_out.at[c, pl.ds(s * WB_FULL, WB_FULL)])

    if with_deg:
        pltpu.sync_copy(hist_v, deg_out.at[wid])


def _make_sc_kernel(with_deg):
    mesh = plsc.VectorSubcoreMesh(core_axis_name="c", subcore_axis_name="s")
    out_type = [pltpu.HBM((NC, N_NODES, D), jnp.float32)]
    if with_deg:
        out_type.append(pltpu.HBM((NW, N_NODES), jnp.float32))
    scratch = [
        pltpu.VMEM((K, CH), jnp.int32),
        pltpu.VMEM((K, CH), jnp.int32),
        pltpu.VMEM((2, CH, D), jnp.float32),
    ]
    if with_deg:
        scratch.append(pltpu.VMEM((N_NODES,), jnp.float32))
    scratch += [
        pltpu.VMEM_SHARED((N_NODES, D), jnp.float32),
        pltpu.SemaphoreType.DMA,
        pltpu.SemaphoreType.DMA,
    ]

    def body(x_hbm, src_hbm, dst_hbm, *rest):
        if with_deg:
            acc_out, deg_out = rest[0], rest[1]
            src_v, dst_v, rows_v, hist_v, acc_sh, sem0, sem1 = rest[2:]
        else:
            acc_out, deg_out = rest[0], None
            src_v, dst_v, rows_v, acc_sh, sem0, sem1 = rest[1:]
            hist_v = None
        _sc_body(x_hbm, src_hbm, dst_hbm, acc_out, deg_out,
                 src_v, dst_v, rows_v, hist_v, acc_sh, sem0, sem1,
                 with_deg=with_deg)

    return pl.kernel(body, out_type=out_type, mesh=mesh, scratch_types=scratch,
                     compiler_params=pltpu.CompilerParams(
                         needs_layout_passes=False,
                         use_tc_tiling_on_sc=False),
                     name="sage_sc_deg" if with_deg else "sage_sc")


_sc_layer1 = _make_sc_kernel(True)
_sc_layer2 = _make_sc_kernel(False)

BR = 400


def _deg_body(degp_ref, out_ref):
    deg = jnp.sum(degp_ref[...], axis=0)
    out_ref[...] = (1.0 / jnp.maximum(deg, 1.0))[:, None]


_deg_reduce = pl.pallas_call(
    _deg_body,
    out_shape=jax.ShapeDtypeStruct((N_NODES, 1), jnp.float32),
)


def _tc_body(acc_ref, dinv_ref, x_ref, wl_ref, bl_ref, wr_ref, out_ref, *, elu):
    agg = (acc_ref[0] + acc_ref[1]) * dinv_ref[...]
    h = (jnp.dot(agg, wl_ref[...], preferred_element_type=jnp.float32)
         + bl_ref[...]
         + jnp.dot(x_ref[...], wr_ref[...], preferred_element_type=jnp.float32))
    if elu:
        h = jnp.where(h > 0, h, jnp.exp(jnp.minimum(h, 0.0)) - 1.0)
    out_ref[...] = h


def _tc_layer(acc, dinv, x, wlT, bl, wrT, elu):
    grid = (N_NODES // BR,)
    return pl.pallas_call(
        functools.partial(_tc_body, elu=elu),
        grid=grid,
        in_specs=[
            pl.BlockSpec((NC, BR, D), lambda i: (0, i, 0)),
            pl.BlockSpec((BR, 1), lambda i: (i, 0)),
            pl.BlockSpec((BR, D), lambda i: (i, 0)),
            pl.BlockSpec((D, D), lambda i: (0, 0)),
            pl.BlockSpec((1, D), lambda i: (0, 0)),
            pl.BlockSpec((D, D), lambda i: (0, 0)),
        ],
        out_specs=pl.BlockSpec((BR, D), lambda i: (i, 0)),
        out_shape=jax.ShapeDtypeStruct((N_NODES, D), jnp.float32),
    )(acc, dinv, x, wlT, bl, wrT)


def kernel(x, edge_index, W1l, b1l, W1r, W2l, b2l, W2r):
    src_r = edge_index[0].astype(jnp.int32).reshape(NW, K, CH)
    dst_r = edge_index[1].astype(jnp.int32).reshape(NW, K, CH)

    acc1, degp = _sc_layer1(x, src_r, dst_r)
    dinv = _deg_reduce(degp)
    h = _tc_layer(acc1, dinv, x, W1l.T, b1l[None, :], W1r.T, elu=True)
    acc2, = _sc_layer2(h, src_r, dst_r)
    out = _tc_layer(acc2, dinv, h, W2l.T, b2l[None, :], W2r.T, elu=False)
    return out


# speedup vs baseline: 12.0992x; 1.0043x over previous
import functools

import jax
import jax.numpy as jnp
from jax import lax
from jax.experimental import pallas as pl
from jax.experimental.pallas import tpu as pltpu
from jax.experimental.pallas import tpu_sc as plsc

N_NODES = 10000
D = 128
N_EDGES = 320000

NC = 2
NS = 16
NW = NC * NS
LANES = 16

CH1 = 80
CH2 = 80
WB = N_NODES // NS
ZC = 25
BR = 400
NB = N_NODES // BR


def _make_sc_kernel(with_deg, ch):
    mesh = plsc.VectorSubcoreMesh(core_axis_name="c", subcore_axis_name="s")
    k = N_EDGES // (NW * ch)
    out_type = [pltpu.HBM((NC, N_NODES, D), jnp.float32)]
    if with_deg:
        out_type.append(pltpu.HBM((NB, NW, BR), jnp.float32))
    scratch = [
        pltpu.VMEM((k, ch), jnp.int32),
        pltpu.VMEM((k, ch), jnp.int32),
        pltpu.VMEM((2, ch, D), jnp.float32),
    ]
    if with_deg:
        scratch.append(pltpu.VMEM((N_NODES,), jnp.float32))
    scratch += [
        pltpu.VMEM_SHARED((N_NODES, D), jnp.float32),
        pltpu.SemaphoreType.DMA,
        pltpu.SemaphoreType.DMA,
    ]

    def body(x_hbm, src_hbm, dst_hbm, *rest):
        if with_deg:
            acc_out, deg_out = rest[0], rest[1]
            src_v, dst_v, rows_v, hist_v, acc_sh, sem0, sem1 = rest[2:]
        else:
            acc_out, deg_out = rest[0], None
            src_v, dst_v, rows_v, acc_sh, sem0, sem1 = rest[1:]
            hist_v = None
        return _sc_body(x_hbm, src_hbm, dst_hbm, acc_out, deg_out,
                        src_v, dst_v, rows_v, hist_v, acc_sh,
                        sem0, sem1, with_deg=with_deg, ch=ch)

    return pl.kernel(body, out_type=out_type, mesh=mesh, scratch_types=scratch,
                     compiler_params=pltpu.CompilerParams(
                         needs_layout_passes=False,
                         use_tc_tiling_on_sc=False),
                     name="sage_sc_deg" if with_deg else "sage_sc")


def _sc_body(x_hbm, src_hbm, dst_hbm, acc_out, deg_out,
             src_v, dst_v, rows_v, hist_v, acc_sh,
             sem0, sem1, *, with_deg, ch):
    k = N_EDGES // (NW * ch)
    c = lax.axis_index("c")
    s = lax.axis_index("s")
    wid = c * NS + s
    sems = (sem0, sem1)

    zeros16 = jnp.zeros((LANES,), jnp.float32)

    @pl.loop(0, 2 * ch)
    def _(r):
        @pl.loop(0, D // LANES)
        def _(g):
            rows_v[r // ch, r % ch, pl.ds(g * LANES, LANES)] = zeros16

    @pl.loop(0, WB // ZC)
    def _(kk):
        pltpu.sync_copy(rows_v.at[0, pl.ds(0, ZC)],
                        acc_sh.at[pl.ds(s * WB + kk * ZC, ZC)])

    if with_deg:
        @pl.loop(0, N_NODES // LANES)
        def _(g):
            hist_v[pl.ds(g * LANES, LANES)] = zeros16

    pltpu.sync_copy(src_hbm.at[wid], src_v)
    pltpu.sync_copy(dst_hbm.at[wid], dst_v)

    plsc.subcore_barrier()

    pltpu.async_copy(x_hbm.at[src_v.at[0]], rows_v.at[0], sems[0])
    pltpu.async_copy(x_hbm.at[src_v.at[1]], rows_v.at[1], sems[1])

    ones16 = jnp.full((LANES,), 1.0, jnp.float32)

    @pl.loop(0, k + (k % 2), step=2)
    def _(j0):
        for b in range(2):
            j = j0 + b

            @pl.when(j < k)
            def _():
                pltpu.make_async_copy(x_hbm.at[src_v.at[j]],
                                      rows_v.at[b], sems[b]).wait()
                pltpu.sync_copy(rows_v.at[b], acc_sh.at[dst_v.at[j]],
                                add=True)
                if with_deg:
                    for g in range(ch // LANES):
                        idx16 = dst_v[j, pl.ds(g * LANES, LANES)]
                        plsc.addupdate_scatter(hist_v, [idx16], ones16)

                @pl.when(j + 2 < k)
                def _():
                    pltpu.async_copy(x_hbm.at[src_v.at[j + 2]],
                                     rows_v.at[b], sems[b])

    plsc.subcore_barrier()

    pltpu.sync_copy(acc_sh.at[pl.ds(s * WB, WB)],
                    acc_out.at[c, pl.ds(s * WB, WB)])
    if with_deg:
        @pl.loop(0, NB)
        def _(blk):
            pltpu.sync_copy(hist_v.at[pl.ds(blk * BR, BR)],
                            deg_out.at[blk, wid])


_sc_layer1 = _make_sc_kernel(True, CH1)
_sc_layer2 = _make_sc_kernel(False, CH2)


def _tc1_body(acc_ref, degp_ref, x_ref, wl_ref, bl_ref, wr_ref,
              out_ref, dinv_ref):
    deg = jnp.sum(degp_ref[0], axis=0)
    dinv = (1.0 / jnp.maximum(deg, 1.0))[:, None]
    dinv_ref[...] = dinv
    agg = (acc_ref[0] + acc_ref[1]) * dinv
    h = (jnp.dot(agg, wl_ref[...], preferred_element_type=jnp.float32)
         + bl_ref[...]
         + jnp.dot(x_ref[...], wr_ref[...], preferred_element_type=jnp.float32))
    out_ref[...] = jnp.where(h > 0, h, jnp.exp(jnp.minimum(h, 0.0)) - 1.0)


_tc_layer1 = pl.pallas_call(
    _tc1_body,
    grid=(NB,),
    in_specs=[
        pl.BlockSpec((NC, BR, D), lambda i: (0, i, 0)),
        pl.BlockSpec((1, NW, BR), lambda i: (i, 0, 0)),
        pl.BlockSpec((BR, D), lambda i: (i, 0)),
        pl.BlockSpec((D, D), lambda i: (0, 0)),
        pl.BlockSpec((1, D), lambda i: (0, 0)),
        pl.BlockSpec((D, D), lambda i: (0, 0)),
    ],
    out_specs=[
        pl.BlockSpec((BR, D), lambda i: (i, 0)),
        pl.BlockSpec((BR, 1), lambda i: (i, 0)),
    ],
    out_shape=[
        jax.ShapeDtypeStruct((N_NODES, D), jnp.float32),
        jax.ShapeDtypeStruct((N_NODES, 1), jnp.float32),
    ],
)


def _tc2_body(acc_ref, dinv_ref, x_ref, wl_ref, bl_ref, wr_ref, out_ref):
    agg = (acc_ref[0] + acc_ref[1]) * dinv_ref[...]
    out_ref[...] = (
        jnp.dot(agg, wl_ref[...], preferred_element_type=jnp.float32)
        + bl_ref[...]
        + jnp.dot(x_ref[...], wr_ref[...], preferred_element_type=jnp.float32))


_tc_layer2 = pl.pallas_call(
    _tc2_body,
    grid=(NB,),
    in_specs=[
        pl.BlockSpec((NC, BR, D), lambda i: (0, i, 0)),
        pl.BlockSpec((BR, 1), lambda i: (i, 0)),
        pl.BlockSpec((BR, D), lambda i: (i, 0)),
        pl.BlockSpec((D, D), lambda i: (0, 0)),
        pl.BlockSpec((1, D), lambda i: (0, 0)),
        pl.BlockSpec((D, D), lambda i: (0, 0)),
    ],
    out_specs=pl.BlockSpec((BR, D), lambda i: (i, 0)),
    out_shape=jax.ShapeDtypeStruct((N_NODES, D), jnp.float32),
)


def kernel(x, edge_index, W1l, b1l, W1r, W2l, b2l, W2r):
    k1 = N_EDGES // (NW * CH1)
    k2 = N_EDGES // (NW * CH2)
    src = edge_index[0].astype(jnp.int32)
    dst = edge_index[1].astype(jnp.int32)

    acc1, degp = _sc_layer1(x, src.reshape(NW, k1, CH1),
                            dst.reshape(NW, k1, CH1))
    h, dinv = _tc_layer1(acc1, degp, x, W1l.T, b1l[None, :], W1r.T)
    acc2, = _sc_layer2(h, src.reshape(NW, k2, CH2),
                       dst.reshape(NW, k2, CH2))
    return _tc_layer2(acc2, dinv, h, W2l.T, b2l[None, :], W2r.T)


# speedup vs baseline: 13.1532x; 1.0871x over previous
import functools

import jax
import jax.numpy as jnp
from jax import lax
from jax.experimental import pallas as pl
from jax.experimental.pallas import tpu as pltpu
from jax.experimental.pallas import tpu_sc as plsc

N_NODES = 10000
D = 128
N_EDGES = 320000

NC = 2
NS = 16
NW = NC * NS
LANES = 16

CH1 = 40
CH2 = 40
NBUF = 4
WB = N_NODES // NS
ZC = 25
BR = 400
NB = N_NODES // BR


def _make_sc_kernel(with_deg, ch):
    mesh = plsc.VectorSubcoreMesh(core_axis_name="c", subcore_axis_name="s")
    k = N_EDGES // (NW * ch)
    out_type = [pltpu.HBM((NC, N_NODES, D), jnp.float32)]
    if with_deg:
        out_type.append(pltpu.HBM((NB, NW, BR), jnp.float32))
    scratch = [
        pltpu.VMEM((k, ch), jnp.int32),
        pltpu.VMEM((k, ch), jnp.int32),
        pltpu.VMEM((NBUF, ch, D), jnp.float32),
    ]
    if with_deg:
        scratch.append(pltpu.VMEM((N_NODES,), jnp.float32))
    scratch += [
        pltpu.VMEM_SHARED((N_NODES, D), jnp.float32),
    ]
    scratch += [pltpu.SemaphoreType.DMA] * (2 * NBUF)

    def body(x_hbm, src_hbm, dst_hbm, *rest):
        if with_deg:
            acc_out, deg_out = rest[0], rest[1]
            src_v, dst_v, rows_v, hist_v, acc_sh = rest[2:7]
            sems = rest[7:]
        else:
            acc_out, deg_out = rest[0], None
            src_v, dst_v, rows_v, acc_sh = rest[1:5]
            sems = rest[5:]
            hist_v = None
        return _sc_body(x_hbm, src_hbm, dst_hbm, acc_out, deg_out,
                        src_v, dst_v, rows_v, hist_v, acc_sh,
                        sems[:NBUF], sems[NBUF:], with_deg=with_deg, ch=ch)

    return pl.kernel(body, out_type=out_type, mesh=mesh, scratch_types=scratch,
                     compiler_params=pltpu.CompilerParams(
                         needs_layout_passes=False,
                         use_tc_tiling_on_sc=False),
                     name="sage_sc_deg" if with_deg else "sage_sc")


def _sc_body(x_hbm, src_hbm, dst_hbm, acc_out, deg_out,
             src_v, dst_v, rows_v, hist_v, acc_sh,
             gsems, ssems, *, with_deg, ch):
    k = N_EDGES // (NW * ch)
    c = lax.axis_index("c")
    s = lax.axis_index("s")
    wid = c * NS + s

    zeros16 = jnp.zeros((LANES,), jnp.float32)

    @pl.loop(0, NBUF * ch)
    def _(r):
        @pl.loop(0, D // LANES)
        def _(g):
            rows_v[r // ch, r % ch, pl.ds(g * LANES, LANES)] = zeros16

    @pl.loop(0, WB // ZC)
    def _(kk):
        pltpu.sync_copy(rows_v.at[0, pl.ds(0, ZC)],
                        acc_sh.at[pl.ds(s * WB + kk * ZC, ZC)])

    if with_deg:
        @pl.loop(0, N_NODES // LANES)
        def _(g):
            hist_v[pl.ds(g * LANES, LANES)] = zeros16

    pltpu.sync_copy(src_hbm.at[wid], src_v)
    pltpu.sync_copy(dst_hbm.at[wid], dst_v)

    plsc.subcore_barrier()

    pltpu.async_copy(x_hbm.at[src_v.at[0]], rows_v.at[0], gsems[0])
    pltpu.async_copy(x_hbm.at[src_v.at[1]], rows_v.at[1], gsems[1])

    ones16 = jnp.full((LANES,), 1.0, jnp.float32)

    @pl.loop(0, k + (-k) % NBUF, step=NBUF)
    def _(j0):
        for b in range(NBUF):
            j = j0 + b
            b2 = (b + 2) % NBUF

            @pl.when(j < k)
            def _():
                @pl.when(jnp.logical_and(j >= 2, j + 2 < k))
                def _():
                    pltpu.make_async_copy(
                        rows_v.at[b2], acc_sh.at[dst_v.at[j - 2]],
                        ssems[b2]).wait()

                @pl.when(j + 2 < k)
                def _():
                    pltpu.async_copy(x_hbm.at[src_v.at[j + 2]],
                                     rows_v.at[b2], gsems[b2])

                pltpu.make_async_copy(x_hbm.at[src_v.at[j]],
                                      rows_v.at[b], gsems[b]).wait()
                pltpu.async_copy(rows_v.at[b], acc_sh.at[dst_v.at[j]],
                                 ssems[b], add=True)
                if with_deg:
                    for g in range(ch // LANES):
                        idx16 = dst_v[j, pl.ds(g * LANES, LANES)]
                        plsc.addupdate_scatter(hist_v, [idx16], ones16)
                    tail = ch - (ch // LANES) * LANES
                    if tail:
                        idx16 = dst_v[j, pl.ds(ch - LANES, LANES)]
                        mask = lax.iota(jnp.int32, LANES) >= (LANES - tail)
                        plsc.addupdate_scatter(hist_v, [idx16], ones16,
                                               mask=mask)

    for r in (4, 3, 2, 1):
        j = k - r
        pltpu.make_async_copy(rows_v.at[j % NBUF], acc_sh.at[dst_v.at[j]],
                              ssems[j % NBUF]).wait()

    plsc.subcore_barrier()

    pltpu.sync_copy(acc_sh.at[pl.ds(s * WB, WB)],
                    acc_out.at[c, pl.ds(s * WB, WB)])
    if with_deg:
        @pl.loop(0, NB)
        def _(blk):
            pltpu.sync_copy(hist_v.at[pl.ds(blk * BR, BR)],
                            deg_out.at[blk, wid])


_sc_layer1 = _make_sc_kernel(True, CH1)
_sc_layer2 = _make_sc_kernel(False, CH2)


def _tc1_body(acc_ref, degp_ref, x_ref, wl_ref, bl_ref, wr_ref,
              out_ref, dinv_ref):
    deg = jnp.sum(degp_ref[0], axis=0)
    dinv = (1.0 / jnp.maximum(deg, 1.0))[:, None]
    dinv_ref[...] = dinv
    agg = (acc_ref[0] + acc_ref[1]) * dinv
    h = (jnp.dot(agg, wl_ref[...], preferred_element_type=jnp.float32)
         + bl_ref[...]
         + jnp.dot(x_ref[...], wr_ref[...], preferred_element_type=jnp.float32))
    out_ref[...] = jnp.where(h > 0, h, jnp.exp(jnp.minimum(h, 0.0)) - 1.0)


_tc_layer1 = pl.pallas_call(
    _tc1_body,
    grid=(NB,),
    in_specs=[
        pl.BlockSpec((NC, BR, D), lambda i: (0, i, 0)),
        pl.BlockSpec((1, NW, BR), lambda i: (i, 0, 0)),
        pl.BlockSpec((BR, D), lambda i: (i, 0)),
        pl.BlockSpec((D, D), lambda i: (0, 0)),
        pl.BlockSpec((1, D), lambda i: (0, 0)),
        pl.BlockSpec((D, D), lambda i: (0, 0)),
    ],
    out_specs=[
        pl.BlockSpec((BR, D), lambda i: (i, 0)),
        pl.BlockSpec((BR, 1), lambda i: (i, 0)),
    ],
    out_shape=[
        jax.ShapeDtypeStruct((N_NODES, D), jnp.float32),
        jax.ShapeDtypeStruct((N_NODES, 1), jnp.float32),
    ],
)


def _tc2_body(acc_ref, dinv_ref, x_ref, wl_ref, bl_ref, wr_ref, out_ref):
    agg = (acc_ref[0] + acc_ref[1]) * dinv_ref[...]
    out_ref[...] = (
        jnp.dot(agg, wl_ref[...], preferred_element_type=jnp.float32)
        + bl_ref[...]
        + jnp.dot(x_ref[...], wr_ref[...], preferred_element_type=jnp.float32))


_tc_layer2 = pl.pallas_call(
    _tc2_body,
    grid=(NB,),
    in_specs=[
        pl.BlockSpec((NC, BR, D), lambda i: (0, i, 0)),
        pl.BlockSpec((BR, 1), lambda i: (i, 0)),
        pl.BlockSpec((BR, D), lambda i: (i, 0)),
        pl.BlockSpec((D, D), lambda i: (0, 0)),
        pl.BlockSpec((1, D), lambda i: (0, 0)),
        pl.BlockSpec((D, D), lambda i: (0, 0)),
    ],
    out_specs=pl.BlockSpec((BR, D), lambda i: (i, 0)),
    out_shape=jax.ShapeDtypeStruct((N_NODES, D), jnp.float32),
)


def kernel(x, edge_index, W1l, b1l, W1r, W2l, b2l, W2r):
    k1 = N_EDGES // (NW * CH1)
    k2 = N_EDGES // (NW * CH2)
    src = edge_index[0].astype(jnp.int32)
    dst = edge_index[1].astype(jnp.int32)

    acc1, degp = _sc_layer1(x, src.reshape(NW, k1, CH1),
                            dst.reshape(NW, k1, CH1))
    h, dinv = _tc_layer1(acc1, degp, x, W1l.T, b1l[None, :], W1r.T)
    acc2, = _sc_layer2(h, src.reshape(NW, k2, CH2),
                       dst.reshape(NW, k2, CH2))
    return _tc_layer2(acc2, dinv, h, W2l.T, b2l[None, :], W2r.T)


# speedup vs baseline: 13.1711x; 1.0014x over previous
import functools

import jax
import jax.numpy as jnp
from jax import lax
from jax.experimental import pallas as pl
from jax.experimental.pallas import tpu as pltpu
from jax.experimental.pallas import tpu_sc as plsc

N_NODES = 10000
D = 128
N_EDGES = 320000

NC = 2
NS = 16
NW = NC * NS
LANES = 16

CH1 = 40
CH2 = 40
NBUF = 4
WB = N_NODES // NS
ZC = 25
BR = 400
NB = N_NODES // BR


def _make_sc_kernel(with_deg, ch):
    mesh = plsc.VectorSubcoreMesh(core_axis_name="c", subcore_axis_name="s")
    k = N_EDGES // (NW * ch)
    out_type = [pltpu.HBM((NC, N_NODES, D), jnp.float32)]
    if with_deg:
        out_type.append(pltpu.HBM((NB, NW, BR), jnp.float32))
    scratch = [
        pltpu.VMEM((k, ch), jnp.int32),
        pltpu.VMEM((k, ch), jnp.int32),
        pltpu.VMEM((NBUF, ch, D), jnp.float32),
    ]
    if with_deg:
        scratch.append(pltpu.VMEM((N_NODES,), jnp.float32))
    scratch += [
        pltpu.VMEM_SHARED((N_NODES, D), jnp.float32),
    ]
    scratch += [pltpu.SemaphoreType.DMA] * (2 * NBUF)

    def body(x_hbm, src_hbm, dst_hbm, *rest):
        if with_deg:
            acc_out, deg_out = rest[0], rest[1]
            src_v, dst_v, rows_v, hist_v, acc_sh = rest[2:7]
            sems = rest[7:]
        else:
            acc_out, deg_out = rest[0], None
            src_v, dst_v, rows_v, acc_sh = rest[1:5]
            sems = rest[5:]
            hist_v = None
        return _sc_body(x_hbm, src_hbm, dst_hbm, acc_out, deg_out,
                        src_v, dst_v, rows_v, hist_v, acc_sh,
                        sems[:NBUF], sems[NBUF:], with_deg=with_deg, ch=ch)

    return pl.kernel(body, out_type=out_type, mesh=mesh, scratch_types=scratch,
                     compiler_params=pltpu.CompilerParams(
                         needs_layout_passes=False,
                         use_tc_tiling_on_sc=False,
                         skip_device_barrier=True),
                     name="sage_sc_deg" if with_deg else "sage_sc")


def _sc_body(x_hbm, src_hbm, dst_hbm, acc_out, deg_out,
             src_v, dst_v, rows_v, hist_v, acc_sh,
             gsems, ssems, *, with_deg, ch):
    k = N_EDGES // (NW * ch)
    c = lax.axis_index("c")
    s = lax.axis_index("s")
    wid = c * NS + s

    zeros16 = jnp.zeros((LANES,), jnp.float32)

    @pl.loop(0, NBUF * ch)
    def _(r):
        @pl.loop(0, D // LANES)
        def _(g):
            rows_v[r // ch, r % ch, pl.ds(g * LANES, LANES)] = zeros16

    @pl.loop(0, WB // ZC)
    def _(kk):
        pltpu.sync_copy(rows_v.at[0, pl.ds(0, ZC)],
                        acc_sh.at[pl.ds(s * WB + kk * ZC, ZC)])

    if with_deg:
        @pl.loop(0, N_NODES // LANES)
        def _(g):
            hist_v[pl.ds(g * LANES, LANES)] = zeros16

    pltpu.sync_copy(src_hbm.at[wid], src_v)
    pltpu.sync_copy(dst_hbm.at[wid], dst_v)

    plsc.subcore_barrier()

    pltpu.async_copy(x_hbm.at[src_v.at[0]], rows_v.at[0], gsems[0])
    pltpu.async_copy(x_hbm.at[src_v.at[1]], rows_v.at[1], gsems[1])

    ones16 = jnp.full((LANES,), 1.0, jnp.float32)

    @pl.loop(0, k + (-k) % NBUF, step=NBUF)
    def _(j0):
        for b in range(NBUF):
            j = j0 + b
            b2 = (b + 2) % NBUF

            @pl.when(j < k)
            def _():
                @pl.when(jnp.logical_and(j >= 2, j + 2 < k))
                def _():
                    pltpu.make_async_copy(
                        rows_v.at[b2], acc_sh.at[dst_v.at[j - 2]],
                        ssems[b2]).wait()

                @pl.when(j + 2 < k)
                def _():
                    pltpu.async_copy(x_hbm.at[src_v.at[j + 2]],
                                     rows_v.at[b2], gsems[b2])

                pltpu.make_async_copy(x_hbm.at[src_v.at[j]],
                                      rows_v.at[b], gsems[b]).wait()
                pltpu.async_copy(rows_v.at[b], acc_sh.at[dst_v.at[j]],
                                 ssems[b], add=True)
                if with_deg:
                    for g in range(ch // LANES):
                        idx16 = dst_v[j, pl.ds(g * LANES, LANES)]
                        plsc.addupdate_scatter(hist_v, [idx16], ones16)
                    tail = ch - (ch // LANES) * LANES
                    if tail:
                        idx16 = dst_v[j, pl.ds(ch - LANES, LANES)]
                        mask = lax.iota(jnp.int32, LANES) >= (LANES - tail)
                        plsc.addupdate_scatter(hist_v, [idx16], ones16,
                                               mask=mask)

    for r in (4, 3, 2, 1):
        j = k - r
        pltpu.make_async_copy(rows_v.at[j % NBUF], acc_sh.at[dst_v.at[j]],
                              ssems[j % NBUF]).wait()

    plsc.subcore_barrier()

    pltpu.sync_copy(acc_sh.at[pl.ds(s * WB, WB)],
                    acc_out.at[c, pl.ds(s * WB, WB)])
    if with_deg:
        @pl.loop(0, NB)
        def _(blk):
            pltpu.sync_copy(hist_v.at[pl.ds(blk * BR, BR)],
                            deg_out.at[blk, wid])


_sc_layer1 = _make_sc_kernel(True, CH1)
_sc_layer2 = _make_sc_kernel(False, CH2)


def _tc1_body(acc_ref, degp_ref, x_ref, wl_ref, bl_ref, wr_ref,
              out_ref, dinv_ref):
    deg = jnp.sum(degp_ref[0], axis=0)
    dinv = (1.0 / jnp.maximum(deg, 1.0))[:, None]
    dinv_ref[...] = dinv
    agg = (acc_ref[0] + acc_ref[1]) * dinv
    h = (jnp.dot(agg, wl_ref[...], preferred_element_type=jnp.float32)
         + bl_ref[...]
         + jnp.dot(x_ref[...], wr_ref[...], preferred_element_type=jnp.float32))
    out_ref[...] = jnp.where(h > 0, h, jnp.exp(jnp.minimum(h, 0.0)) - 1.0)


_tc_layer1 = pl.pallas_call(
    _tc1_body,
    grid=(NB,),
    in_specs=[
        pl.BlockSpec((NC, BR, D), lambda i: (0, i, 0)),
        pl.BlockSpec((1, NW, BR), lambda i: (i, 0, 0)),
        pl.BlockSpec((BR, D), lambda i: (i, 0)),
        pl.BlockSpec((D, D), lambda i: (0, 0)),
        pl.BlockSpec((1, D), lambda i: (0, 0)),
        pl.BlockSpec((D, D), lambda i: (0, 0)),
    ],
    out_specs=[
        pl.BlockSpec((BR, D), lambda i: (i, 0)),
        pl.BlockSpec((BR, 1), lambda i: (i, 0)),
    ],
    out_shape=[
        jax.ShapeDtypeStruct((N_NODES, D), jnp.float32),
        jax.ShapeDtypeStruct((N_NODES, 1), jnp.float32),
    ],
)


def _tc2_body(acc_ref, dinv_ref, x_ref, wl_ref, bl_ref, wr_ref, out_ref):
    agg = (acc_ref[0] + acc_ref[1]) * dinv_ref[...]
    out_ref[...] = (
        jnp.dot(agg, wl_ref[...], preferred_element_type=jnp.float32)
        + bl_ref[...]
        + jnp.dot(x_ref[...], wr_ref[...], preferred_element_type=jnp.float32))


_tc_layer2 = pl.pallas_call(
    _tc2_body,
    grid=(NB,),
    in_specs=[
        pl.BlockSpec((NC, BR, D), lambda i: (0, i, 0)),
        pl.BlockSpec((BR, 1), lambda i: (i, 0)),
        pl.BlockSpec((BR, D), lambda i: (i, 0)),
        pl.BlockSpec((D, D), lambda i: (0, 0)),
        pl.BlockSpec((1, D), lambda i: (0, 0)),
        pl.BlockSpec((D, D), lambda i: (0, 0)),
    ],
    out_specs=pl.BlockSpec((BR, D), lambda i: (i, 0)),
    out_shape=jax.ShapeDtypeStruct((N_NODES, D), jnp.float32),
)


def kernel(x, edge_index, W1l, b1l, W1r, W2l, b2l, W2r):
    k1 = N_EDGES // (NW * CH1)
    k2 = N_EDGES // (NW * CH2)
    src = edge_index[0].astype(jnp.int32)
    dst = edge_index[1].astype(jnp.int32)

    acc1, degp = _sc_layer1(x, src.reshape(NW, k1, CH1),
                            dst.reshape(NW, k1, CH1))
    h, dinv = _tc_layer1(acc1, degp, x, W1l.T, b1l[None, :], W1r.T)
    acc2, = _sc_layer2(h, src.reshape(NW, k2, CH2),
                       dst.reshape(NW, k2, CH2))
    return _tc_layer2(acc2, dinv, h, W2l.T, b2l[None, :], W2r.T)


# speedup vs baseline: 13.5312x; 1.0273x over previous
import functools

import jax
import jax.numpy as jnp
from jax import lax
from jax.experimental import pallas as pl
from jax.experimental.pallas import tpu as pltpu
from jax.experimental.pallas import tpu_sc as plsc

N_NODES = 10000
D = 128
N_EDGES = 320000

NC = 2
NS = 16
NW = NC * NS
LANES = 16

CH1 = 40
CH2 = 40
NBUF1 = 4
NBUF2 = 5
WB = N_NODES // NS
ZC = 25
BR = 400
NB = N_NODES // BR


def _make_sc_kernel(with_deg, ch, nbuf):
    mesh = plsc.VectorSubcoreMesh(core_axis_name="c", subcore_axis_name="s")
    k = N_EDGES // (NW * ch)
    out_type = [pltpu.HBM((NC, N_NODES, D), jnp.float32)]
    if with_deg:
        out_type.append(pltpu.HBM((NB, NW, BR), jnp.float32))
    scratch = [
        pltpu.VMEM((k, ch), jnp.int32),
        pltpu.VMEM((k, ch), jnp.int32),
        pltpu.VMEM((nbuf, ch, D), jnp.float32),
    ]
    if with_deg:
        scratch.append(pltpu.VMEM((N_NODES,), jnp.float32))
    scratch += [
        pltpu.VMEM_SHARED((N_NODES, D), jnp.float32),
    ]
    scratch += [pltpu.SemaphoreType.DMA] * (2 * nbuf)

    def body(x_hbm, src_hbm, dst_hbm, *rest):
        if with_deg:
            acc_out, deg_out = rest[0], rest[1]
            src_v, dst_v, rows_v, hist_v, acc_sh = rest[2:7]
            sems = rest[7:]
        else:
            acc_out, deg_out = rest[0], None
            src_v, dst_v, rows_v, acc_sh = rest[1:5]
            sems = rest[5:]
            hist_v = None
        return _sc_body(x_hbm, src_hbm, dst_hbm, acc_out, deg_out,
                        src_v, dst_v, rows_v, hist_v, acc_sh,
                        sems[:nbuf], sems[nbuf:], with_deg=with_deg, ch=ch,
                        nbuf=nbuf)

    return pl.kernel(body, out_type=out_type, mesh=mesh, scratch_types=scratch,
                     compiler_params=pltpu.CompilerParams(
                         needs_layout_passes=False,
                         use_tc_tiling_on_sc=False),
                     name="sage_sc_deg" if with_deg else "sage_sc")


def _sc_body(x_hbm, src_hbm, dst_hbm, acc_out, deg_out,
             src_v, dst_v, rows_v, hist_v, acc_sh,
             gsems, ssems, *, with_deg, ch, nbuf):
    k = N_EDGES // (NW * ch)
    glead = nbuf - 2
    c = lax.axis_index("c")
    s = lax.axis_index("s")
    wid = c * NS + s

    zeros16 = jnp.zeros((LANES,), jnp.float32)

    @pl.loop(0, nbuf * ch)
    def _(r):
        @pl.loop(0, D // LANES)
        def _(g):
            rows_v[r // ch, r % ch, pl.ds(g * LANES, LANES)] = zeros16

    @pl.loop(0, WB // ZC)
    def _(kk):
        pltpu.sync_copy(rows_v.at[0, pl.ds(0, ZC)],
                        acc_sh.at[pl.ds(s * WB + kk * ZC, ZC)])

    if with_deg:
        @pl.loop(0, N_NODES // LANES)
        def _(g):
            hist_v[pl.ds(g * LANES, LANES)] = zeros16

    pltpu.sync_copy(src_hbm.at[wid], src_v)
    pltpu.sync_copy(dst_hbm.at[wid], dst_v)

    plsc.subcore_barrier()

    for p in range(nbuf - 2):
        pltpu.async_copy(x_hbm.at[src_v.at[p]], rows_v.at[p], gsems[p])

    ones16 = jnp.full((LANES,), 1.0, jnp.float32)

    @pl.loop(0, k + (-k) % nbuf, step=nbuf)
    def _(j0):
        for b in range(nbuf):
            j = j0 + b
            b2 = (b + glead) % nbuf

            @pl.when(j < k)
            def _():
                @pl.when(jnp.logical_and(j >= 2, j + glead < k))
                def _():
                    pltpu.make_async_copy(
                        rows_v.at[b2], acc_sh.at[dst_v.at[j - 2]],
                        ssems[b2]).wait()

                @pl.when(j + glead < k)
                def _():
                    pltpu.async_copy(x_hbm.at[src_v.at[j + glead]],
                                     rows_v.at[b2], gsems[b2])

                pltpu.make_async_copy(x_hbm.at[src_v.at[j]],
                                      rows_v.at[b], gsems[b]).wait()
                pltpu.async_copy(rows_v.at[b], acc_sh.at[dst_v.at[j]],
                                 ssems[b], add=True)
                if with_deg:
                    for g in range(ch // LANES):
                        idx16 = dst_v[j, pl.ds(g * LANES, LANES)]
                        plsc.addupdate_scatter(hist_v, [idx16], ones16)
                    tail = ch - (ch // LANES) * LANES
                    if tail:
                        idx16 = dst_v[j, pl.ds(ch - LANES, LANES)]
                        mask = lax.iota(jnp.int32, LANES) >= (LANES - tail)
                        plsc.addupdate_scatter(hist_v, [idx16], ones16,
                                               mask=mask)

    for r in range(nbuf, 0, -1):
        j = k - r
        pltpu.make_async_copy(rows_v.at[j % nbuf], acc_sh.at[dst_v.at[j]],
                              ssems[j % nbuf]).wait()

    plsc.subcore_barrier()

    pltpu.sync_copy(acc_sh.at[pl.ds(s * WB, WB)],
                    acc_out.at[c, pl.ds(s * WB, WB)])
    if with_deg:
        @pl.loop(0, NB)
        def _(blk):
            pltpu.sync_copy(hist_v.at[pl.ds(blk * BR, BR)],
                            deg_out.at[blk, wid])


_sc_layer1 = _make_sc_kernel(True, CH1, NBUF1)
_sc_layer2 = _make_sc_kernel(False, CH2, NBUF2)


def _tc1_body(acc_ref, degp_ref, x_ref, wl_ref, bl_ref, wr_ref,
              out_ref, dinv_ref):
    deg = jnp.sum(degp_ref[0], axis=0)
    dinv = (1.0 / jnp.maximum(deg, 1.0))[:, None]
    dinv_ref[...] = dinv
    agg = (acc_ref[0] + acc_ref[1]) * dinv
    h = (jnp.dot(agg, wl_ref[...], preferred_element_type=jnp.float32)
         + bl_ref[...]
         + jnp.dot(x_ref[...], wr_ref[...], preferred_element_type=jnp.float32))
    out_ref[...] = jnp.where(h > 0, h, jnp.exp(jnp.minimum(h, 0.0)) - 1.0)


_tc_layer1 = pl.pallas_call(
    _tc1_body,
    grid=(NB,),
    in_specs=[
        pl.BlockSpec((NC, BR, D), lambda i: (0, i, 0)),
        pl.BlockSpec((1, NW, BR), lambda i: (i, 0, 0)),
        pl.BlockSpec((BR, D), lambda i: (i, 0)),
        pl.BlockSpec((D, D), lambda i: (0, 0)),
        pl.BlockSpec((1, D), lambda i: (0, 0)),
        pl.BlockSpec((D, D), lambda i: (0, 0)),
    ],
    out_specs=[
        pl.BlockSpec((BR, D), lambda i: (i, 0)),
        pl.BlockSpec((BR, 1), lambda i: (i, 0)),
    ],
    out_shape=[
        jax.ShapeDtypeStruct((N_NODES, D), jnp.float32),
        jax.ShapeDtypeStruct((N_NODES, 1), jnp.float32),
    ],
)


def _tc2_body(acc_ref, dinv_ref, x_ref, wl_ref, bl_ref, wr_ref, out_ref):
    agg = (acc_ref[0] + acc_ref[1]) * dinv_ref[...]
    out_ref[...] = (
        jnp.dot(agg, wl_ref[...], preferred_element_type=jnp.float32)
        + bl_ref[...]
        + jnp.dot(x_ref[...], wr_ref[...], preferred_element_type=jnp.float32))


_tc_layer2 = pl.pallas_call(
    _tc2_body,
    grid=(NB,),
    in_specs=[
        pl.BlockSpec((NC, BR, D), lambda i: (0, i, 0)),
        pl.BlockSpec((BR, 1), lambda i: (i, 0)),
        pl.BlockSpec((BR, D), lambda i: (i, 0)),
        pl.BlockSpec((D, D), lambda i: (0, 0)),
        pl.BlockSpec((1, D), lambda i: (0, 0)),
        pl.BlockSpec((D, D), lambda i: (0, 0)),
    ],
    out_specs=pl.BlockSpec((BR, D), lambda i: (i, 0)),
    out_shape=jax.ShapeDtypeStruct((N_NODES, D), jnp.float32),
)


def kernel(x, edge_index, W1l, b1l, W1r, W2l, b2l, W2r):
    k1 = N_EDGES // (NW * CH1)
    k2 = N_EDGES // (NW * CH2)
    src = edge_index[0].astype(jnp.int32)
    dst = edge_index[1].astype(jnp.int32)

    acc1, degp = _sc_layer1(x, src.reshape(NW, k1, CH1),
                            dst.reshape(NW, k1, CH1))
    h, dinv = _tc_layer1(acc1, degp, x, W1l.T, b1l[None, :], W1r.T)
    acc2, = _sc_layer2(h, src.reshape(NW, k2, CH2),
                       dst.reshape(NW, k2, CH2))
    return _tc_layer2(acc2, dinv, h, W2l.T, b2l[None, :], W2r.T)
